# baseline (speedup 1.0000x reference)
import functools

import jax
import jax.numpy as jnp
from jax import lax
from jax.experimental import pallas as pl
from jax.experimental.pallas import tpu as pltpu
from jax.experimental.pallas import tpu_sc as plsc

_K = 8192
_D = 256
_M = 16384
_BM = 512
_BN = 1024
_NI = _M // _BM
_NJ = _K // _BN
_CC = 0.25

_WB0, _WB1 = 2736, 5472


def _round_bf16(v):
    bits = jax.lax.bitcast_convert_type(v, jnp.int32)
    lsb = jax.lax.shift_right_logical(bits, 16) & 1
    rounded = (bits + 0x7FFF + lsb) & jnp.int32(-65536)
    return jax.lax.bitcast_convert_type(rounded, jnp.float32)


def _masked_min(d2, mask, iota):
    inf = jnp.float32(jnp.inf)
    m = jnp.min(jnp.where(mask, d2, inf), axis=1, keepdims=True)
    i = jnp.min(jnp.where(mask & (d2 <= m), iota, 2**30),
                axis=1, keepdims=True)
    return m, i


def _argmin_body(x_ref, et_ref, a2_ref, b2_ref, idx_ref, lsum_ref,
                 win_m, win_i, glob_m, glob_i, glob_lv):
    i = pl.program_id(0)
    j = pl.program_id(1)

    x = x_ref[...]
    et = et_ref[...]
    xyn = jnp.dot(x, et, preferred_element_type=jnp.float32)
    d2 = (a2_ref[...] + b2_ref[...]) + xyn

    iota_l = lax.broadcasted_iota(jnp.int32, d2.shape, 1)
    giota = iota_l + j * _BN
    is_boundary = (j == 2) | (j == 5)

    @pl.when(jnp.logical_not(is_boundary))
    def _plain_chunk():
        m_c = jnp.min(d2, axis=1, keepdims=True)
        i_c = jnp.argmin(d2, axis=1).astype(jnp.int32)[:, None] + j * _BN

        @pl.when(j == 0)
        def _init():
            win_m[...] = m_c
            win_i[...] = i_c
            glob_m[...] = jnp.full_like(m_c, jnp.inf)
            glob_i[...] = jnp.zeros_like(i_c)
            glob_lv[...] = jnp.zeros_like(m_c)

        @pl.when(j > 0)
        def _merge():
            better = m_c < win_m[...]
            win_i[...] = jnp.where(better, i_c, win_i[...])
            win_m[...] = jnp.where(better, m_c, win_m[...])

    @pl.when(is_boundary)
    def _boundary_chunk():
        s = jnp.where(j == 2, _WB0 - 2 * _BN, _WB1 - 5 * _BN)
        in_lo = iota_l < s
        m_lo, i_lo = _masked_min(d2, in_lo, giota)
        better = m_lo < win_m[...]
        fold_m = jnp.where(better, m_lo, win_m[...])
        fold_i = jnp.where(better, i_lo, win_i[...])
        upd = fold_m < glob_m[...]
        glob_i[...] = jnp.where(upd, fold_i, glob_i[...])
        glob_lv[...] = jnp.where(upd, fold_m, glob_lv[...])
        glob_m[...] = jnp.where(upd, _round_bf16(fold_m), glob_m[...])
        m_hi, i_hi = _masked_min(d2, jnp.logical_not(in_lo), giota)
        win_m[...] = m_hi
        win_i[...] = i_hi

    @pl.when(j == _NJ - 1)
    def _fold_last():
        upd = win_m[...] < glob_m[...]
        glob_i[...] = jnp.where(upd, win_i[...], glob_i[...])
        glob_lv[...] = jnp.where(upd, win_m[...], glob_lv[...])
        glob_m[...] = jnp.where(upd, _round_bf16(win_m[...]), glob_m[...])

    @pl.when(j == _NJ - 1)
    def _finish():
        idx_ref[...] = glob_i[...]
        @pl.when(i == 0)
        def _zero():
            lsum_ref[0, 0] = 0.0
        lsum_ref[0, 0] += jnp.sum(glob_lv[...])


_argmin_call = pl.pallas_call(
    _argmin_body,
    grid=(_NI, _NJ),
    in_specs=[
        pl.BlockSpec((_BM, _D), lambda i, j: (i, 0)),
        pl.BlockSpec((_D, _BN), lambda i, j: (0, j)),
        pl.BlockSpec((_BM, 1), lambda i, j: (i, 0)),
        pl.BlockSpec((1, _BN), lambda i, j: (0, j)),
    ],
    out_specs=[
        pl.BlockSpec((_BM, 1), lambda i, j: (i, 0)),
        pl.BlockSpec(memory_space=pltpu.SMEM),
    ],
    out_shape=[
        jax.ShapeDtypeStruct((_M, 1), jnp.int32),
        jax.ShapeDtypeStruct((1, 1), jnp.float32),
    ],
    scratch_shapes=[
        pltpu.VMEM((_BM, 1), jnp.float32),
        pltpu.VMEM((_BM, 1), jnp.int32),
        pltpu.VMEM((_BM, 1), jnp.float32),
        pltpu.VMEM((_BM, 1), jnp.int32),
        pltpu.VMEM((_BM, 1), jnp.float32),
    ],
)


_CH = 128


@functools.cache
def _get_sc_gather():
    info = plsc.get_sparse_core_info()
    nc, ns = info.num_cores, info.num_subcores
    nw = nc * ns
    bpw = _M // nw
    nchunk = bpw // _CH

    @functools.partial(
        pl.kernel,
        mesh=plsc.VectorSubcoreMesh(core_axis_name="c", subcore_axis_name="s"),
        out_type=jax.ShapeDtypeStruct((_M, _D), jnp.float32),
        scratch_types=[
            pltpu.VMEM((_CH,), jnp.int32),
            pltpu.VMEM((_CH, _D), jnp.float32),
            pltpu.SemaphoreType.DMA,
        ],
    )
    def _sc_gather(table_hbm, idx_hbm, out_hbm, idx_v, rows_v, sem):
        wid = lax.axis_index("s") * nc + lax.axis_index("c")
        base = wid * bpw

        def chunk(k, carry):
            off = base + k * _CH
            pltpu.sync_copy(idx_hbm.at[pl.ds(off, _CH)], idx_v)
            pltpu.async_copy(table_hbm.at[idx_v], rows_v, sem).wait()
            pltpu.sync_copy(rows_v, out_hbm.at[pl.ds(off, _CH)])
            return carry

        lax.fori_loop(0, nchunk, chunk, 0)

    return _sc_gather


def kernel(inputs, embed):
    inputs = inputs.astype(jnp.float32)
    channel_last = jnp.transpose(inputs, (0, 2, 3, 4, 1))
    input_shape = channel_last.shape
    x = channel_last.reshape(-1, _D)

    a2 = jnp.sum(x * x, axis=1, keepdims=True)
    b2 = jnp.sum(embed * embed, axis=1)[None, :]
    et = embed.T * jnp.float32(-2.0)

    idx2d, lsum = _argmin_call(x, et, a2, b2)
    idx = idx2d[:, 0]

    q = _get_sc_gather()(embed, idx)
    quantized = q.reshape(input_shape)
    quantized = jnp.transpose(quantized, (0, 4, 1, 2, 3))

    loss = _CC * (lsum[0, 0] / jnp.float32(_M * _D))
    quantized_st = inputs + (quantized - inputs)
    encoding_indices = idx.reshape(input_shape[:-1])
    return loss, quantized_st, encoding_indices


# speedup vs baseline: 1.4465x; 1.4465x over previous
import functools

import jax
import jax.numpy as jnp
from jax import lax
from jax.experimental import pallas as pl
from jax.experimental.pallas import tpu as pltpu
from jax.experimental.pallas import tpu_sc as plsc

_K = 8192
_D = 256
_M = 16384
_BM = 1024
_BN = 2048
_NI = _M // _BM
_NJ = _K // _BN
_CC = 0.25

_WB0, _WB1 = 2736, 5472


def _round_bf16(v):
    bits = jax.lax.bitcast_convert_type(v, jnp.int32)
    lsb = jax.lax.shift_right_logical(bits, 16) & 1
    rounded = (bits + 0x7FFF + lsb) & jnp.int32(-65536)
    return jax.lax.bitcast_convert_type(rounded, jnp.float32)


def _masked_min(d2, mask, iota):
    inf = jnp.float32(jnp.inf)
    m = jnp.min(jnp.where(mask, d2, inf), axis=1, keepdims=True)
    i = jnp.min(jnp.where(mask & (d2 <= m), iota, 2**30),
                axis=1, keepdims=True)
    return m, i


def _argmin_body(x_ref, et_ref, a2_ref, b2_ref, idx_ref, lsum_ref,
                 win_m, win_i, glob_m, glob_i, glob_lv):
    i = pl.program_id(0)
    j = pl.program_id(1)

    x = x_ref[...]
    et = et_ref[...]
    xyn = jnp.dot(x, et, preferred_element_type=jnp.float32)
    d2 = (a2_ref[...] + b2_ref[...]) + xyn

    iota_l = lax.broadcasted_iota(jnp.int32, d2.shape, 1)
    c0 = j * _BN
    giota = iota_l + c0
    has_b0 = (c0 < _WB0) & (_WB0 < c0 + _BN)
    has_b1 = (c0 < _WB1) & (_WB1 < c0 + _BN)
    is_boundary = has_b0 | has_b1

    @pl.when(jnp.logical_not(is_boundary))
    def _plain_chunk():
        m_c = jnp.min(d2, axis=1, keepdims=True)
        i_c = jnp.min(jnp.where(d2 <= m_c, giota, 2**30),
                      axis=1, keepdims=True)

        @pl.when(j == 0)
        def _init():
            win_m[...] = m_c
            win_i[...] = i_c
            glob_m[...] = jnp.full_like(m_c, jnp.inf)
            glob_i[...] = jnp.zeros_like(i_c)
            glob_lv[...] = jnp.zeros_like(m_c)

        @pl.when(j > 0)
        def _merge():
            better = m_c < win_m[...]
            win_i[...] = jnp.where(better, i_c, win_i[...])
            win_m[...] = jnp.where(better, m_c, win_m[...])

    @pl.when(is_boundary)
    def _boundary_chunk():
        s = jnp.where(has_b0, _WB0 - c0, _WB1 - c0)
        in_lo = iota_l < s
        m_lo, i_lo = _masked_min(d2, in_lo, giota)
        better = m_lo < win_m[...]
        fold_m = jnp.where(better, m_lo, win_m[...])
        fold_i = jnp.where(better, i_lo, win_i[...])
        upd = fold_m < glob_m[...]
        glob_i[...] = jnp.where(upd, fold_i, glob_i[...])
        glob_lv[...] = jnp.where(upd, fold_m, glob_lv[...])
        glob_m[...] = jnp.where(upd, _round_bf16(fold_m), glob_m[...])
        m_hi, i_hi = _masked_min(d2, jnp.logical_not(in_lo), giota)
        win_m[...] = m_hi
        win_i[...] = i_hi

    @pl.when(j == _NJ - 1)
    def _fold_last():
        upd = win_m[...] < glob_m[...]
        glob_i[...] = jnp.where(upd, win_i[...], glob_i[...])
        glob_lv[...] = jnp.where(upd, win_m[...], glob_lv[...])
        glob_m[...] = jnp.where(upd, _round_bf16(win_m[...]), glob_m[...])

    @pl.when(j == _NJ - 1)
    def _finish():
        idx_ref[...] = glob_i[...]
        @pl.when(i == 0)
        def _zero():
            lsum_ref[0, 0] = 0.0
        lsum_ref[0, 0] += jnp.sum(glob_lv[...])


_argmin_call = pl.pallas_call(
    _argmin_body,
    grid=(_NI, _NJ),
    in_specs=[
        pl.BlockSpec((_BM, _D), lambda i, j: (i, 0)),
        pl.BlockSpec((_D, _BN), lambda i, j: (0, j)),
        pl.BlockSpec((_BM, 1), lambda i, j: (i, 0)),
        pl.BlockSpec((1, _BN), lambda i, j: (0, j)),
    ],
    out_specs=[
        pl.BlockSpec((_BM, 1), lambda i, j: (i, 0)),
        pl.BlockSpec(memory_space=pltpu.SMEM),
    ],
    out_shape=[
        jax.ShapeDtypeStruct((_M, 1), jnp.int32),
        jax.ShapeDtypeStruct((1, 1), jnp.float32),
    ],
    scratch_shapes=[
        pltpu.VMEM((_BM, 1), jnp.float32),
        pltpu.VMEM((_BM, 1), jnp.int32),
        pltpu.VMEM((_BM, 1), jnp.float32),
        pltpu.VMEM((_BM, 1), jnp.int32),
        pltpu.VMEM((_BM, 1), jnp.float32),
    ],
)


_CH = 128


@functools.cache
def _get_sc_gather():
    info = plsc.get_sparse_core_info()
    nc, ns = info.num_cores, info.num_subcores
    nw = nc * ns
    bpw = _M // nw
    nchunk = bpw // _CH

    @functools.partial(
        pl.kernel,
        mesh=plsc.VectorSubcoreMesh(core_axis_name="c", subcore_axis_name="s"),
        out_type=jax.ShapeDtypeStruct((_M, _D), jnp.float32),
        scratch_types=[
            pltpu.VMEM((_CH,), jnp.int32),
            pltpu.VMEM((_CH, _D), jnp.float32),
            pltpu.SemaphoreType.DMA,
        ],
    )
    def _sc_gather(table_hbm, idx_hbm, out_hbm, idx_v, rows_v, sem):
        wid = lax.axis_index("s") * nc + lax.axis_index("c")
        base = wid * bpw

        def chunk(k, carry):
            off = base + k * _CH
            pltpu.sync_copy(idx_hbm.at[pl.ds(off, _CH)], idx_v)
            pltpu.async_copy(table_hbm.at[idx_v], rows_v, sem).wait()
            pltpu.sync_copy(rows_v, out_hbm.at[pl.ds(off, _CH)])
            return carry

        lax.fori_loop(0, nchunk, chunk, 0)

    return _sc_gather


def kernel(inputs, embed):
    inputs = inputs.astype(jnp.float32)
    channel_last = jnp.transpose(inputs, (0, 2, 3, 4, 1))
    input_shape = channel_last.shape
    x = channel_last.reshape(-1, _D)

    a2 = jnp.sum(x * x, axis=1, keepdims=True)
    b2 = jnp.sum(embed * embed, axis=1)[None, :]
    et = embed.T * jnp.float32(-2.0)

    idx2d, lsum = _argmin_call(x, et, a2, b2)
    idx = idx2d[:, 0]

    q = _get_sc_gather()(embed, idx)
    quantized = q.reshape(input_shape)
    quantized = jnp.transpose(quantized, (0, 4, 1, 2, 3))

    loss = _CC * (lsum[0, 0] / jnp.float32(_M * _D))
    quantized_st = inputs + (quantized - inputs)
    encoding_indices = idx.reshape(input_shape[:-1])
    return loss, quantized_st, encoding_indices


# speedup vs baseline: 1.4988x; 1.0362x over previous
import functools

import jax
import jax.numpy as jnp
from jax import lax
from jax.experimental import pallas as pl
from jax.experimental.pallas import tpu as pltpu
from jax.experimental.pallas import tpu_sc as plsc

_K = 8192
_D = 256
_M = 16384
_BM = 2048
_BN = 2048
_NI = _M // _BM
_NJ = _K // _BN
_CC = 0.25

_WB0, _WB1 = 2736, 5472


def _round_bf16(v):
    bits = jax.lax.bitcast_convert_type(v, jnp.int32)
    lsb = jax.lax.shift_right_logical(bits, 16) & 1
    rounded = (bits + 0x7FFF + lsb) & jnp.int32(-65536)
    return jax.lax.bitcast_convert_type(rounded, jnp.float32)


def _masked_min(d2, mask, iota):
    inf = jnp.float32(jnp.inf)
    m = jnp.min(jnp.where(mask, d2, inf), axis=1, keepdims=True)
    i = jnp.min(jnp.where(mask & (d2 <= m), iota, 2**30),
                axis=1, keepdims=True)
    return m, i


def _argmin_body(x_ref, et_ref, a2_ref, b2_ref, idx_ref, lsum_ref,
                 win_m, win_i, glob_m, glob_i, glob_lv):
    i = pl.program_id(0)
    j = pl.program_id(1)

    x = x_ref[...]
    et = et_ref[...]
    xyn = jnp.dot(x, et, preferred_element_type=jnp.float32)
    d2 = (a2_ref[...] + b2_ref[...]) + xyn

    iota_l = lax.broadcasted_iota(jnp.int32, d2.shape, 1)
    c0 = j * _BN
    giota = iota_l + c0
    has_b0 = (c0 < _WB0) & (_WB0 < c0 + _BN)
    has_b1 = (c0 < _WB1) & (_WB1 < c0 + _BN)
    is_boundary = has_b0 | has_b1

    @pl.when(jnp.logical_not(is_boundary))
    def _plain_chunk():
        m_c = jnp.min(d2, axis=1, keepdims=True)
        i_c = jnp.min(jnp.where(d2 <= m_c, giota, 2**30),
                      axis=1, keepdims=True)

        @pl.when(j == 0)
        def _init():
            win_m[...] = m_c
            win_i[...] = i_c
            glob_m[...] = jnp.full_like(m_c, jnp.inf)
            glob_i[...] = jnp.zeros_like(i_c)
            glob_lv[...] = jnp.zeros_like(m_c)

        @pl.when(j > 0)
        def _merge():
            better = m_c < win_m[...]
            win_i[...] = jnp.where(better, i_c, win_i[...])
            win_m[...] = jnp.where(better, m_c, win_m[...])

    @pl.when(is_boundary)
    def _boundary_chunk():
        s = jnp.where(has_b0, _WB0 - c0, _WB1 - c0)
        in_lo = iota_l < s
        m_lo, i_lo = _masked_min(d2, in_lo, giota)
        better = m_lo < win_m[...]
        fold_m = jnp.where(better, m_lo, win_m[...])
        fold_i = jnp.where(better, i_lo, win_i[...])
        upd = fold_m < glob_m[...]
        glob_i[...] = jnp.where(upd, fold_i, glob_i[...])
        glob_lv[...] = jnp.where(upd, fold_m, glob_lv[...])
        glob_m[...] = jnp.where(upd, _round_bf16(fold_m), glob_m[...])
        m_hi, i_hi = _masked_min(d2, jnp.logical_not(in_lo), giota)
        win_m[...] = m_hi
        win_i[...] = i_hi

    @pl.when(j == _NJ - 1)
    def _fold_last():
        upd = win_m[...] < glob_m[...]
        glob_i[...] = jnp.where(upd, win_i[...], glob_i[...])
        glob_lv[...] = jnp.where(upd, win_m[...], glob_lv[...])
        glob_m[...] = jnp.where(upd, _round_bf16(win_m[...]), glob_m[...])

    @pl.when(j == _NJ - 1)
    def _finish():
        idx_ref[...] = glob_i[...]
        @pl.when(i == 0)
        def _zero():
            lsum_ref[0, 0] = 0.0
        lsum_ref[0, 0] += jnp.sum(glob_lv[...])


_argmin_call = pl.pallas_call(
    _argmin_body,
    grid=(_NI, _NJ),
    in_specs=[
        pl.BlockSpec((_BM, _D), lambda i, j: (i, 0)),
        pl.BlockSpec((_D, _BN), lambda i, j: (0, j)),
        pl.BlockSpec((_BM, 1), lambda i, j: (i, 0)),
        pl.BlockSpec((1, _BN), lambda i, j: (0, j)),
    ],
    out_specs=[
        pl.BlockSpec((_BM, 1), lambda i, j: (i, 0)),
        pl.BlockSpec(memory_space=pltpu.SMEM),
    ],
    out_shape=[
        jax.ShapeDtypeStruct((_M, 1), jnp.int32),
        jax.ShapeDtypeStruct((1, 1), jnp.float32),
    ],
    scratch_shapes=[
        pltpu.VMEM((_BM, 1), jnp.float32),
        pltpu.VMEM((_BM, 1), jnp.int32),
        pltpu.VMEM((_BM, 1), jnp.float32),
        pltpu.VMEM((_BM, 1), jnp.int32),
        pltpu.VMEM((_BM, 1), jnp.float32),
    ],
)


_CH = 128


@functools.cache
def _get_sc_gather():
    info = plsc.get_sparse_core_info()
    nc, ns = info.num_cores, info.num_subcores
    nw = nc * ns
    bpw = _M // nw
    nchunk = bpw // _CH

    @functools.partial(
        pl.kernel,
        mesh=plsc.VectorSubcoreMesh(core_axis_name="c", subcore_axis_name="s"),
        out_type=jax.ShapeDtypeStruct((_M, _D), jnp.float32),
        scratch_types=[
            pltpu.VMEM((_CH,), jnp.int32),
            pltpu.VMEM((_CH, _D), jnp.float32),
            pltpu.SemaphoreType.DMA,
        ],
    )
    def _sc_gather(table_hbm, idx_hbm, out_hbm, idx_v, rows_v, sem):
        wid = lax.axis_index("s") * nc + lax.axis_index("c")
        base = wid * bpw

        def chunk(k, carry):
            off = base + k * _CH
            pltpu.sync_copy(idx_hbm.at[pl.ds(off, _CH)], idx_v)
            pltpu.async_copy(table_hbm.at[idx_v], rows_v, sem).wait()
            pltpu.sync_copy(rows_v, out_hbm.at[pl.ds(off, _CH)])
            return carry

        lax.fori_loop(0, nchunk, chunk, 0)

    return _sc_gather


def kernel(inputs, embed):
    inputs = inputs.astype(jnp.float32)
    channel_last = jnp.transpose(inputs, (0, 2, 3, 4, 1))
    input_shape = channel_last.shape
    x = channel_last.reshape(-1, _D)

    a2 = jnp.sum(x * x, axis=1, keepdims=True)
    b2 = jnp.sum(embed * embed, axis=1)[None, :]
    et = embed.T * jnp.float32(-2.0)

    idx2d, lsum = _argmin_call(x, et, a2, b2)
    idx = idx2d[:, 0]

    q = _get_sc_gather()(embed, idx)
    quantized = q.reshape(input_shape)
    quantized = jnp.transpose(quantized, (0, 4, 1, 2, 3))

    loss = _CC * (lsum[0, 0] / jnp.float32(_M * _D))
    quantized_st = inputs + (quantized - inputs)
    encoding_indices = idx.reshape(input_shape[:-1])
    return loss, quantized_st, encoding_indices
